# baseline (device time: 31916 ns/iter reference)
import jax
import jax.numpy as jnp
from jax import lax
from jax.experimental import pallas as pl
from jax.experimental.pallas import tpu as pltpu

N_DEV = 4
E = 32
E_LOCAL = 8
CAP = 51
CAP_PAD = 64
N_TOK = 2048
D = 512
H = 1024
M = N_TOK // N_DEV
S = E_LOCAL * CAP_PAD
CAPC = 192

BF = jnp.bfloat16


def _fused_body(slotr_ref, rkr_ref, cols_ref, x_ref, w_hbm_ref,
                out_ref, w_ref, yg_ref, snd_ref, comm_ref,
                wcopy_sem, send_sems, recv_sems):
    my = lax.axis_index("i")
    left = jnp.mod(my - 1, N_DEV)
    right = jnp.mod(my + 1, N_DEV)
    diag = jnp.mod(my + 2, N_DEV)

    wcopy = pltpu.make_async_copy(w_hbm_ref, w_ref, wcopy_sem)
    wcopy.start()

    barrier_sem = pltpu.get_barrier_semaphore()
    for nbr in (left, right, diag):
        pl.semaphore_signal(
            barrier_sem, inc=1,
            device_id=(nbr,), device_id_type=pl.DeviceIdType.MESH,
        )
    pl.semaphore_wait(barrier_sem, 3)

    iota_s = lax.broadcasted_iota(jnp.int32, (S, N_TOK), 0)
    P = (iota_s == slotr_ref[:, :]).astype(BF)
    xg = jnp.dot(P, x_ref[:, :], preferred_element_type=jnp.float32)

    wcopy.wait()
    for e in range(E_LOCAL):
        yg_ref[e * CAP_PAD:(e + 1) * CAP_PAD, :] = jnp.dot(
            xg[e * CAP_PAD:(e + 1) * CAP_PAD, :], w_ref[e],
            preferred_element_type=jnp.float32,
        ).astype(BF)
    yg = yg_ref[:, :]

    def compact_for(c):
        rk_c = rkr_ref[pl.ds(c, 1), :]
        rct = (lax.broadcasted_iota(jnp.int32, (CAPC, M), 0)
               == rk_c).astype(BF)
        slot_c = cols_ref[pl.ds(c * M, M), 0:1]
        pct = (lax.broadcasted_iota(jnp.int32, (M, S), 1)
               == slot_c).astype(BF)
        cc = jnp.dot(rct, pct,
                     preferred_element_type=jnp.float32).astype(BF)
        return jnp.dot(cc, yg,
                       preferred_element_type=jnp.float32).astype(BF)

    dsts = (diag, right, left)
    slots = (1, 0, 2)
    rdmas = []
    for (dst, sl) in zip(dsts, slots):
        snd_ref[sl] = compact_for(dst)
        r = pltpu.make_async_remote_copy(
            src_ref=snd_ref.at[sl], dst_ref=comm_ref.at[sl],
            send_sem=send_sems.at[sl], recv_sem=recv_sems.at[sl],
            device_id=(dst,), device_id_type=pl.DeviceIdType.MESH,
        )
        r.start()
        rdmas.append(r)

    slot_m = cols_ref[pl.ds(my * M, M), 0:1]
    pmt = (lax.broadcasted_iota(jnp.int32, (M, S), 1) == slot_m).astype(BF)
    acc = jnp.dot(pmt, yg, preferred_element_type=jnp.float32)

    rk_m = cols_ref[pl.ds(my * M, M), 1:2]
    own_m = cols_ref[pl.ds(my * M, M), 2:3]
    iota_c = lax.broadcasted_iota(jnp.int32, (M, CAPC), 1)
    for sl, k in ((0, 1), (2, 3), (1, 2)):
        src = jnp.mod(my - k, N_DEV)
        rdmas[slots.index(sl)].wait_recv()
        q = ((own_m == src) & (rk_m == iota_c)).astype(BF)
        acc = acc + jnp.dot(q, comm_ref[sl],
                            preferred_element_type=jnp.float32)
    out_ref[:, :] = acc
    for r in rdmas:
        r.wait_send()


def _tok_cumsum(a):
    k = a.shape[1]
    b = a.reshape(16, 128, k)
    within = jnp.cumsum(b, axis=1)
    totals = within[:, -1, :]
    prefix = jnp.cumsum(totals, axis=0) - totals
    return (within + prefix[:, None, :]).reshape(N_TOK, k)


def kernel(x, router_W, route_idx, expert_W):
    del router_W
    my = lax.axis_index("i")

    e = route_idx[:, 0].astype(jnp.int32)
    onehot = (e[:, None] == jnp.arange(E, dtype=jnp.int32)[None, :])
    cum = _tok_cumsum(onehot.astype(jnp.int32))
    pos = cum[:, E - 1]
    for j in range(E - 1):
        pos = jnp.where(e == j, cum[:, j], pos)
    pos = pos - 1
    keep = pos < CAP
    owner = e // E_LOCAL
    el = e - E_LOCAL * my
    mine = (owner == my) & keep
    slot = jnp.where(mine, el * CAP_PAD + pos, S).astype(jnp.int32)

    oh4 = (owner[:, None] == jnp.arange(N_DEV, dtype=jnp.int32)[None, :]) \
        & keep[:, None]
    cum4 = _tok_cumsum(oh4.astype(jnp.int32))
    cb = cum4.reshape(N_DEV, M, N_DEV)
    base = jnp.concatenate(
        [jnp.zeros((1, N_DEV), jnp.int32), cb[:-1, -1, :]], axis=0)
    blockcum = (cb - base[:, None, :]).reshape(N_TOK, N_DEV)
    rkv = blockcum[:, N_DEV - 1]
    for j in range(N_DEV - 1):
        rkv = jnp.where(owner == j, blockcum[:, j], rkv)
    rkv = jnp.where(keep, rkv - 1, -1).astype(jnp.int32)

    cols = jnp.stack([slot, rkv, owner.astype(jnp.int32)], axis=1)

    return pl.pallas_call(
        _fused_body,
        out_shape=jax.ShapeDtypeStruct((M, H), jnp.float32),
        in_specs=[pl.BlockSpec(memory_space=pltpu.VMEM)] * 4
        + [pl.BlockSpec(memory_space=pl.ANY)],
        out_specs=pl.BlockSpec(memory_space=pltpu.VMEM),
        scratch_shapes=[
            pltpu.VMEM((E_LOCAL, D, H), jnp.float32),
            pltpu.VMEM((S, H), BF),
            pltpu.VMEM((N_DEV - 1, CAPC, H), BF),
            pltpu.VMEM((N_DEV - 1, CAPC, H), BF),
            pltpu.SemaphoreType.DMA(()),
            pltpu.SemaphoreType.DMA((N_DEV - 1,)),
            pltpu.SemaphoreType.DMA((N_DEV - 1,)),
        ],
        compiler_params=pltpu.CompilerParams(collective_id=0),
    )(
        slot[None, :],
        rkv.reshape(N_DEV, M),
        cols,
        x.astype(BF),
        expert_W,
    )
